# baseline (speedup 1.0000x reference)
import functools

import jax
import jax.numpy as jnp
from jax import lax
from jax.experimental import pallas as pl
from jax.experimental.pallas import tpu as pltpu
from jax.experimental.pallas import tpu_sc as plsc

N = 10000
E = 320000
D = 128
G = 64
NC = 2
NS = 16
CH = 160
K = 128
EP = NS * CH * K
SB = 16
NSB = CH // SB
NPAD = 10240
NN = 2 * N

_f32 = jnp.float32


@functools.cache
def _make_sc_degree():
    mesh = plsc.VectorSubcoreMesh(core_axis_name="c", subcore_axis_name="s")
    return functools.partial(
        pl.kernel,
        out_type=jax.ShapeDtypeStruct((NC, 1, N * 16), _f32),
        mesh=mesh,
        scratch_types=[
            pltpu.VMEM((SB, 1, K), jnp.int32),
            pltpu.VMEM((K,), _f32),
            pltpu.VMEM((2000,), _f32),
            pltpu.VMEM_SHARED((NPAD * 16,), _f32),
        ],
    )(_sc_degree_body)


def _sc_degree_body(cols_hbm, deg_hbm, col_v, ones_v, zer_v, acc):
    cid = lax.axis_index("c")
    sid = lax.axis_index("s")
    for i in range(K // 16):
        ones_v[pl.ds(i * 16, 16)] = jnp.ones((16,), _f32)

    def _z(i, c):
        zer_v[pl.ds(i * 16, 16)] = jnp.zeros((16,), _f32)
        return c

    lax.fori_loop(0, 2000 // 16, _z, 0)

    @pl.when(sid < 10)
    def _():
        for t in range(8):
            pltpu.sync_copy(zer_v, acc.at[pl.ds(sid * 16000 + t * 2000, 2000)])

    plsc.subcore_barrier()
    base = (cid * NS + sid) * CH

    def _body(s, c):
        pltpu.sync_copy(cols_hbm.at[pl.ds(base + s * SB, SB)], col_v)
        for m in range(SB):
            pltpu.sync_copy(ones_v, acc.at[col_v.at[m, 0]], add=True)
        return c

    lax.fori_loop(0, NSB, _body, 0)
    plsc.subcore_barrier()

    @pl.when(sid < 10)
    def _():
        pltpu.sync_copy(acc.at[pl.ds(sid * 16000, 16000)],
                        deg_hbm.at[cid, 0, pl.ds(sid * 16000, 16000)])


@functools.cache
def _make_sc_scatter():
    mesh = plsc.VectorSubcoreMesh(core_axis_name="c", subcore_axis_name="s")
    return functools.partial(
        pl.kernel,
        out_type=jax.ShapeDtypeStruct((NC, N, D), _f32),
        mesh=mesh,
        scratch_types=[
            pltpu.VMEM((SB, 2, K), jnp.int32),
            pltpu.VMEM((K, D), _f32),
            pltpu.VMEM((K, D), _f32),
            pltpu.VMEM_SHARED((NPAD, D), _f32),
            pltpu.SemaphoreType.DMA,
            pltpu.SemaphoreType.DMA,
            pltpu.SemaphoreType.DMA,
            pltpu.SemaphoreType.DMA,
        ],
    )(_sc_scatter_body)


def _sc_scatter_body(y_hbm, rc_hbm, out_hbm, rc_v, buf0, buf1, acc,
                     sem0, sem1, sems0, sems1):
    cid = lax.axis_index("c")
    sid = lax.axis_index("s")

    def _z(i, c):
        for k in range(D // 16):
            buf0[i, pl.ds(k * 16, 16)] = jnp.zeros((16,), _f32)
        return c

    lax.fori_loop(0, K, _z, 0)

    @pl.when(sid < 10)
    def _():
        for t in range(8):
            pltpu.sync_copy(buf0.at[pl.ds(0, 125)],
                            acc.at[pl.ds(sid * 1000 + t * 125, 125)])

    plsc.subcore_barrier()
    base = (cid * NS + sid) * CH
    bufs = (buf0, buf1)
    semg = (sem0, sem1)
    semsc = (sems0, sems1)

    def _body(s, c):
        pltpu.sync_copy(rc_hbm.at[pl.ds(base + s * SB, SB)], rc_v)
        pltpu.async_copy(y_hbm.at[rc_v.at[0, 0]], buf0, sem0)
        pltpu.async_copy(y_hbm.at[rc_v.at[1, 0]], buf1, sem1)
        for m in range(SB):
            b, sg = bufs[m % 2], semg[m % 2]
            pltpu.make_async_copy(y_hbm.at[rc_v.at[m, 0]], b, sg).wait()
            pltpu.async_copy(b, acc.at[rc_v.at[m, 1]], semsc[m % 2], add=True)
            if m >= 1:
                bp, sgp = bufs[(m - 1) % 2], semg[(m - 1) % 2]
                pltpu.make_async_copy(
                    bp, acc.at[rc_v.at[m - 1, 1]], semsc[(m - 1) % 2]).wait()
                if m + 1 < SB:
                    pltpu.async_copy(y_hbm.at[rc_v.at[m + 1, 0]], bp, sgp)
        pltpu.make_async_copy(buf1, acc.at[rc_v.at[SB - 1, 1]],
                              sems1).wait()
        return c

    lax.fori_loop(0, NSB, _body, 0)
    plsc.subcore_barrier()

    @pl.when(sid < 10)
    def _():
        pltpu.sync_copy(acc.at[pl.ds(sid * 1000, 1000)],
                        out_hbm.at[cid, pl.ds(sid * 1000, 1000)])


def _tc_scale_matmul(x_ref, w_ref, deg16_ref, y_ref, dinv_ref):
    deg = jnp.sum(deg16_ref[...], axis=1, keepdims=True) + 1.0
    dinv = lax.rsqrt(deg)
    xw = jnp.dot(x_ref[...], w_ref[...], preferred_element_type=_f32)
    y_ref[...] = xw * dinv
    dinv_ref[...] = dinv


def _tc_mid(s_ref, y_ref, dinv_ref, b_ref, w2_ref, y2_ref):
    dinv = dinv_ref[...]
    h = jnp.maximum(dinv * (s_ref[...] + y_ref[...]) + b_ref[...], 0.0)
    y2_ref[...] = jnp.dot(h, w2_ref[...], preferred_element_type=_f32) * dinv


def _tc_final(s_ref, y_ref, dinv_ref, b_ref, bat1_ref, bat2_ref,
              wfc_ref, bfc_ref, o_ref):
    h = jnp.maximum(dinv_ref[...] * (s_ref[...] + y_ref[...]) + b_ref[...], 0.0)

    def _pool(hb, bat):
        onehot = (lax.broadcasted_iota(jnp.int32, (G, N), 0) == bat).astype(_f32)
        pooled = jnp.dot(onehot, hb, preferred_element_type=_f32)
        cnt = jnp.maximum(jnp.sum(onehot, axis=1, keepdims=True), 1.0)
        return pooled / cnt

    g1 = _pool(h[:N], bat1_ref[...])
    g2 = _pool(h[N:], bat2_ref[...])
    z = jnp.concatenate([g1, g2], axis=1)
    logit = jnp.dot(z, wfc_ref[...], preferred_element_type=_f32) + bfc_ref[0, 0]
    o_ref[...] = jax.nn.sigmoid(logit)


def kernel(x1, edge_index1, batch1, x2, edge_index2, batch2,
           W1, b1, W2, b2, Wfc, bfc):
    ei1 = edge_index1.astype(jnp.int32)
    ei2 = edge_index2.astype(jnp.int32)
    npad = EP - E
    pid = jnp.arange(npad, dtype=jnp.int32)
    pad_rows = pid % 4096
    pad_cols = N + pid % (NPAD - N)
    eid = jnp.arange(EP, dtype=jnp.int32)

    def prep(ei, base):
        rows = jnp.concatenate([ei[0] + base, pad_rows + base])
        cols = jnp.concatenate([ei[1], pad_cols])
        cols16 = cols * 16 + eid % 16
        rc = jnp.stack([rows.reshape(NS * CH, K),
                        cols.reshape(NS * CH, K)], axis=1)
        return rc, cols16.reshape(NS * CH, 1, K)

    rc1, h1 = prep(ei1, 0)
    rc2, h2 = prep(ei2, N)
    rc_sc = jnp.concatenate([rc1, rc2])
    cols16_sc = jnp.concatenate([h1, h2])

    deg16 = _make_sc_degree()(cols16_sc).reshape(NN, 16)

    x_flat = jnp.concatenate([x1, x2], axis=0)
    y1, dinv = pl.pallas_call(
        _tc_scale_matmul,
        out_shape=(jax.ShapeDtypeStruct((NN, D), _f32),
                   jax.ShapeDtypeStruct((NN, 1), _f32)),
    )(x_flat, W1, deg16)

    s1 = _make_sc_scatter()(y1, rc_sc).reshape(NN, D)

    y2 = pl.pallas_call(
        _tc_mid,
        out_shape=jax.ShapeDtypeStruct((NN, D), _f32),
    )(s1, y1, dinv, b1.reshape(1, D), W2)

    s2 = _make_sc_scatter()(y2, rc_sc).reshape(NN, D)

    out = pl.pallas_call(
        _tc_final,
        out_shape=jax.ShapeDtypeStruct((G, 1), _f32),
    )(s2, y2, dinv, b2.reshape(1, D),
      batch1.astype(jnp.int32).reshape(1, N),
      batch2.astype(jnp.int32).reshape(1, N),
      Wfc, bfc.reshape(1, 1))
    return out


# speedup vs baseline: 1.1442x; 1.1442x over previous
import functools

import jax
import jax.numpy as jnp
from jax import lax
from jax.experimental import pallas as pl
from jax.experimental.pallas import tpu as pltpu
from jax.experimental.pallas import tpu_sc as plsc

N = 10000
E = 320000
D = 128
G = 64
NC = 2
NS = 16
CH = 160
K = 128
EP = NS * CH * K
SB = 16
NSB = CH // SB
NPAD = 10240
NN = 2 * N

_f32 = jnp.float32


@functools.cache
def _make_sc_degree():
    mesh = plsc.VectorSubcoreMesh(core_axis_name="c", subcore_axis_name="s")
    return functools.partial(
        pl.kernel,
        out_type=jax.ShapeDtypeStruct((NC, 1, N * 16), _f32),
        mesh=mesh,
        scratch_types=[
            pltpu.VMEM((SB, 1, K), jnp.int32),
            pltpu.VMEM((K,), _f32),
            pltpu.VMEM((2000,), _f32),
            pltpu.VMEM_SHARED((NPAD * 16,), _f32),
        ],
    )(_sc_degree_body)


def _sc_degree_body(cols_hbm, deg_hbm, col_v, ones_v, zer_v, acc):
    cid = lax.axis_index("c")
    sid = lax.axis_index("s")
    for i in range(K // 16):
        ones_v[pl.ds(i * 16, 16)] = jnp.ones((16,), _f32)

    def _z(i, c):
        zer_v[pl.ds(i * 16, 16)] = jnp.zeros((16,), _f32)
        return c

    lax.fori_loop(0, 2000 // 16, _z, 0)

    @pl.when(sid < 10)
    def _():
        for t in range(8):
            pltpu.sync_copy(zer_v, acc.at[pl.ds(sid * 16000 + t * 2000, 2000)])

    plsc.subcore_barrier()
    base = (cid * NS + sid) * CH

    def _body(s, c):
        pltpu.sync_copy(cols_hbm.at[pl.ds(base + s * SB, SB)], col_v)
        for m in range(SB):
            pltpu.sync_copy(ones_v, acc.at[col_v.at[m, 0]], add=True)
        return c

    lax.fori_loop(0, NSB, _body, 0)
    plsc.subcore_barrier()

    @pl.when(sid < 10)
    def _():
        pltpu.sync_copy(acc.at[pl.ds(sid * 16000, 16000)],
                        deg_hbm.at[cid, 0, pl.ds(sid * 16000, 16000)])


@functools.cache
def _make_sc_scatter():
    mesh = plsc.VectorSubcoreMesh(core_axis_name="c", subcore_axis_name="s")
    return functools.partial(
        pl.kernel,
        out_type=jax.ShapeDtypeStruct((NC, N, D), _f32),
        mesh=mesh,
        scratch_types=[
            pltpu.VMEM((SB, 2, K), jnp.int32),
            pltpu.VMEM((K, D), _f32),
            pltpu.VMEM((K, D), _f32),
            pltpu.VMEM_SHARED((NPAD, D), _f32),
            pltpu.SemaphoreType.DMA,
            pltpu.SemaphoreType.DMA,
        ],
    )(_sc_scatter_body)


def _sc_scatter_body(y_hbm, rc_hbm, out_hbm, rc_v, buf0, buf1, acc,
                     sem0, sem1):
    cid = lax.axis_index("c")
    sid = lax.axis_index("s")

    @pl.when(sid < 10)
    def _():
        pltpu.sync_copy(y_hbm.at[pl.ds(cid * N + sid * 1000, 1000)],
                        acc.at[pl.ds(sid * 1000, 1000)])

    plsc.subcore_barrier()
    base = (cid * NS + sid) * CH
    bufs = (buf0, buf1)
    sems = (sem0, sem1)

    def _body(s, c):
        pltpu.sync_copy(rc_hbm.at[pl.ds(base + s * SB, SB)], rc_v)
        pltpu.async_copy(y_hbm.at[rc_v.at[0, 0]], buf0, sem0)
        pltpu.async_copy(y_hbm.at[rc_v.at[1, 0]], buf1, sem1)
        for m in range(SB):
            b, sm = bufs[m % 2], sems[m % 2]
            pltpu.make_async_copy(y_hbm.at[rc_v.at[m, 0]], b, sm).wait()
            pltpu.sync_copy(b, acc.at[rc_v.at[m, 1]], add=True)
            if m + 2 < SB:
                pltpu.async_copy(y_hbm.at[rc_v.at[m + 2, 0]], b, sm)
        return c

    lax.fori_loop(0, NSB, _body, 0)
    plsc.subcore_barrier()

    @pl.when(sid < 10)
    def _():
        pltpu.sync_copy(acc.at[pl.ds(sid * 1000, 1000)],
                        out_hbm.at[cid, pl.ds(sid * 1000, 1000)])


def _tc_matmul(x_ref, w_ref, xw_ref):
    xw_ref[...] = jnp.dot(x_ref[...], w_ref[...], preferred_element_type=_f32)


def _tc_scale(xw_ref, deg16_ref, y_ref, dinv_ref):
    deg = jnp.sum(deg16_ref[...], axis=1, keepdims=True) + 1.0
    dinv = lax.rsqrt(deg)
    y_ref[...] = xw_ref[...] * dinv
    dinv_ref[...] = dinv


def _tc_mid(s_ref, dinv_ref, b_ref, w2_ref, y2_ref):
    dinv = dinv_ref[...]
    h = jnp.maximum(dinv * s_ref[...] + b_ref[...], 0.0)
    y2_ref[...] = jnp.dot(h, w2_ref[...], preferred_element_type=_f32) * dinv


def _tc_final(s_ref, dinv_ref, b_ref, bat1_ref, bat2_ref,
              wfc_ref, bfc_ref, o_ref):
    h = jnp.maximum(dinv_ref[...] * s_ref[...] + b_ref[...], 0.0)

    def _pool(hb, bat):
        onehot = (lax.broadcasted_iota(jnp.int32, (G, N), 0) == bat).astype(_f32)
        pooled = jnp.dot(onehot, hb, preferred_element_type=_f32)
        cnt = jnp.maximum(jnp.sum(onehot, axis=1, keepdims=True), 1.0)
        return pooled / cnt

    g1 = _pool(h[:N], bat1_ref[...])
    g2 = _pool(h[N:], bat2_ref[...])
    z = jnp.concatenate([g1, g2], axis=1)
    logit = jnp.dot(z, wfc_ref[...], preferred_element_type=_f32) + bfc_ref[0, 0]
    o_ref[...] = jax.nn.sigmoid(logit)


def kernel(x1, edge_index1, batch1, x2, edge_index2, batch2,
           W1, b1, W2, b2, Wfc, bfc):
    ei1 = edge_index1.astype(jnp.int32)
    ei2 = edge_index2.astype(jnp.int32)
    npad = EP - E
    pid = jnp.arange(npad, dtype=jnp.int32)
    pad_rows = pid % 4096
    pad_cols = N + pid % (NPAD - N)
    eid = jnp.arange(EP, dtype=jnp.int32)

    def prep(ei, base):
        rows = jnp.concatenate([ei[0] + base, pad_rows + base])
        cols = jnp.concatenate([ei[1], pad_cols])
        cols16 = cols * 16 + eid % 16
        rc = jnp.stack([rows.reshape(NS * CH, K),
                        cols.reshape(NS * CH, K)], axis=1)
        return rc, cols16.reshape(NS * CH, 1, K)

    rc1, h1 = prep(ei1, 0)
    rc2, h2 = prep(ei2, N)
    rc_sc = jnp.concatenate([rc1, rc2])
    cols16_sc = jnp.concatenate([h1, h2])

    x_flat = jnp.concatenate([x1, x2], axis=0)
    xw = pl.pallas_call(
        _tc_matmul,
        out_shape=jax.ShapeDtypeStruct((NN, D), _f32),
    )(x_flat, W1)
    deg16 = _make_sc_degree()(cols16_sc).reshape(NN, 16)

    y1, dinv = pl.pallas_call(
        _tc_scale,
        out_shape=(jax.ShapeDtypeStruct((NN, D), _f32),
                   jax.ShapeDtypeStruct((NN, 1), _f32)),
    )(xw, deg16)

    s1 = _make_sc_scatter()(y1, rc_sc).reshape(NN, D)

    y2 = pl.pallas_call(
        _tc_mid,
        out_shape=jax.ShapeDtypeStruct((NN, D), _f32),
    )(s1, dinv, b1.reshape(1, D), W2)

    s2 = _make_sc_scatter()(y2, rc_sc).reshape(NN, D)

    out = pl.pallas_call(
        _tc_final,
        out_shape=jax.ShapeDtypeStruct((G, 1), _f32),
    )(s2, dinv, b2.reshape(1, D),
      batch1.astype(jnp.int32).reshape(1, N),
      batch2.astype(jnp.int32).reshape(1, N),
      Wfc, bfc.reshape(1, 1))
    return out


# speedup vs baseline: 1.1549x; 1.0094x over previous
import functools

import jax
import jax.numpy as jnp
from jax import lax
from jax.experimental import pallas as pl
from jax.experimental.pallas import tpu as pltpu
from jax.experimental.pallas import tpu_sc as plsc

N = 10000
E = 320000
D = 128
G = 64
NC = 2
NS = 16
CH = 160
K = 128
EP = NS * CH * K
SB = 16
NSB = CH // SB
NPAD = 10240
NN = 2 * N

_f32 = jnp.float32


@functools.cache
def _make_sc_degree():
    mesh = plsc.VectorSubcoreMesh(core_axis_name="c", subcore_axis_name="s")
    return functools.partial(
        pl.kernel,
        out_type=jax.ShapeDtypeStruct((NC, 1, N * 16), _f32),
        mesh=mesh,
        scratch_types=[
            pltpu.VMEM((SB, 1, K), jnp.int32),
            pltpu.VMEM((K,), _f32),
            pltpu.VMEM((2000,), _f32),
            pltpu.VMEM_SHARED((NPAD * 16,), _f32),
        ],
    )(_sc_degree_body)


def _sc_degree_body(cols_hbm, deg_hbm, col_v, ones_v, zer_v, acc):
    cid = lax.axis_index("c")
    sid = lax.axis_index("s")
    for i in range(K // 16):
        ones_v[pl.ds(i * 16, 16)] = jnp.ones((16,), _f32)

    def _z(i, c):
        zer_v[pl.ds(i * 16, 16)] = jnp.zeros((16,), _f32)
        return c

    lax.fori_loop(0, 2000 // 16, _z, 0)

    @pl.when(sid < 10)
    def _():
        for t in range(8):
            pltpu.sync_copy(zer_v, acc.at[pl.ds(sid * 16000 + t * 2000, 2000)])

    plsc.subcore_barrier()
    base = (cid * NS + sid) * CH

    def _body(s, c):
        pltpu.sync_copy(cols_hbm.at[pl.ds(base + s * SB, SB)], col_v)
        for m in range(SB):
            pltpu.sync_copy(ones_v, acc.at[col_v.at[m, 0]], add=True)
        return c

    lax.fori_loop(0, NSB, _body, 0)
    plsc.subcore_barrier()

    @pl.when(sid < 10)
    def _():
        pltpu.sync_copy(acc.at[pl.ds(sid * 16000, 16000)],
                        deg_hbm.at[cid, 0, pl.ds(sid * 16000, 16000)])


@functools.cache
def _make_sc_scatter():
    mesh = plsc.VectorSubcoreMesh(core_axis_name="c", subcore_axis_name="s")
    return functools.partial(
        pl.kernel,
        out_type=jax.ShapeDtypeStruct((NC, N, D), _f32),
        mesh=mesh,
        scratch_types=[
            pltpu.VMEM((SB, 2, K), jnp.int32),
            pltpu.VMEM((K, D), _f32),
            pltpu.VMEM((K, D), _f32),
            pltpu.VMEM_SHARED((NPAD, D), _f32),
            pltpu.SemaphoreType.DMA,
            pltpu.SemaphoreType.DMA,
        ],
    )(_sc_scatter_body)


def _sc_scatter_body(y_hbm, rc_hbm, out_hbm, rc_v, buf0, buf1, acc,
                     sem0, sem1):
    cid = lax.axis_index("c")
    sid = lax.axis_index("s")

    @pl.when(sid < 10)
    def _():
        pltpu.sync_copy(y_hbm.at[pl.ds(cid * N + sid * 1000, 1000)],
                        acc.at[pl.ds(sid * 1000, 1000)])

    plsc.subcore_barrier()
    base = (cid * NS + sid) * CH
    bufs = (buf0, buf1)
    sems = (sem0, sem1)

    def _body(s, c):
        pltpu.sync_copy(rc_hbm.at[pl.ds(base + s * SB, SB)], rc_v)
        pltpu.async_copy(y_hbm.at[rc_v.at[0, 0]], buf0, sem0)
        pltpu.async_copy(y_hbm.at[rc_v.at[1, 0]], buf1, sem1)
        for m in range(SB):
            b, sm = bufs[m % 2], sems[m % 2]
            pltpu.make_async_copy(y_hbm.at[rc_v.at[m, 0]], b, sm).wait()
            pltpu.sync_copy(b, acc.at[rc_v.at[m, 1]], add=True)
            if m + 2 < SB:
                pltpu.async_copy(y_hbm.at[rc_v.at[m + 2, 0]], b, sm)
        return c

    lax.fori_loop(0, NSB, _body, 0)
    plsc.subcore_barrier()

    @pl.when(sid < 10)
    def _():
        pltpu.sync_copy(acc.at[pl.ds(sid * 1000, 1000)],
                        out_hbm.at[cid, pl.ds(sid * 1000, 1000)])


RB = 2000


def _tc_scale_matmul(x_ref, w_ref, deg16_ref, y_ref, dinv_ref):
    deg = jnp.sum(deg16_ref[...], axis=1, keepdims=True) + 1.0
    dinv = lax.rsqrt(deg)
    xw = jnp.dot(x_ref[...], w_ref[...], preferred_element_type=_f32)
    y_ref[...] = xw * dinv
    dinv_ref[...] = dinv


def _tc_mid(s_ref, dinv_ref, b_ref, w2_ref, y2_ref):
    dinv = dinv_ref[...]
    h = jnp.maximum(dinv * s_ref[...] + b_ref[...], 0.0)
    y2_ref[...] = jnp.dot(h, w2_ref[...], preferred_element_type=_f32) * dinv


def _tc_final(s_ref, dinv_ref, b_ref, bat1_ref, bat2_ref,
              wfc_ref, bfc_ref, o_ref):
    h = jnp.maximum(dinv_ref[...] * s_ref[...] + b_ref[...], 0.0)

    def _pool(hb, bat):
        onehot = (lax.broadcasted_iota(jnp.int32, (G, N), 0) == bat).astype(_f32)
        pooled = jnp.dot(onehot, hb, preferred_element_type=_f32)
        cnt = jnp.maximum(jnp.sum(onehot, axis=1, keepdims=True), 1.0)
        return pooled / cnt

    g1 = _pool(h[:N], bat1_ref[...])
    g2 = _pool(h[N:], bat2_ref[...])
    z = jnp.concatenate([g1, g2], axis=1)
    logit = jnp.dot(z, wfc_ref[...], preferred_element_type=_f32) + bfc_ref[0, 0]
    o_ref[...] = jax.nn.sigmoid(logit)


def kernel(x1, edge_index1, batch1, x2, edge_index2, batch2,
           W1, b1, W2, b2, Wfc, bfc):
    ei1 = edge_index1.astype(jnp.int32)
    ei2 = edge_index2.astype(jnp.int32)
    npad = EP - E
    pid = jnp.arange(npad, dtype=jnp.int32)
    pad_rows = pid % 4096
    pad_cols = N + pid % (NPAD - N)
    eid = jnp.arange(EP, dtype=jnp.int32)

    def prep(ei, base):
        rows = jnp.concatenate([ei[0] + base, pad_rows + base])
        cols = jnp.concatenate([ei[1], pad_cols])
        cols16 = cols * 16 + eid % 16
        rc = jnp.stack([rows.reshape(NS * CH, K),
                        cols.reshape(NS * CH, K)], axis=1)
        return rc, cols16.reshape(NS * CH, 1, K)

    rc1, h1 = prep(ei1, 0)
    rc2, h2 = prep(ei2, N)
    rc_sc = jnp.concatenate([rc1, rc2])
    cols16_sc = jnp.concatenate([h1, h2])

    deg16 = _make_sc_degree()(cols16_sc).reshape(NN, 16)

    x_flat = jnp.concatenate([x1, x2], axis=0)
    nb = NN // RB
    y1, dinv = pl.pallas_call(
        _tc_scale_matmul,
        grid=(nb,),
        in_specs=[pl.BlockSpec((RB, D), lambda i: (i, 0)),
                  pl.BlockSpec((D, D), lambda i: (0, 0)),
                  pl.BlockSpec((RB, 16), lambda i: (i, 0))],
        out_specs=(pl.BlockSpec((RB, D), lambda i: (i, 0)),
                   pl.BlockSpec((RB, 1), lambda i: (i, 0))),
        out_shape=(jax.ShapeDtypeStruct((NN, D), _f32),
                   jax.ShapeDtypeStruct((NN, 1), _f32)),
    )(x_flat, W1, deg16)

    s1 = _make_sc_scatter()(y1, rc_sc).reshape(NN, D)

    y2 = pl.pallas_call(
        _tc_mid,
        grid=(nb,),
        in_specs=[pl.BlockSpec((RB, D), lambda i: (i, 0)),
                  pl.BlockSpec((RB, 1), lambda i: (i, 0)),
                  pl.BlockSpec((1, D), lambda i: (0, 0)),
                  pl.BlockSpec((D, D), lambda i: (0, 0))],
        out_specs=pl.BlockSpec((RB, D), lambda i: (i, 0)),
        out_shape=jax.ShapeDtypeStruct((NN, D), _f32),
    )(s1, dinv, b1.reshape(1, D), W2)

    s2 = _make_sc_scatter()(y2, rc_sc).reshape(NN, D)

    out = pl.pallas_call(
        _tc_final,
        out_shape=jax.ShapeDtypeStruct((G, 1), _f32),
    )(s2, dinv, b2.reshape(1, D),
      batch1.astype(jnp.int32).reshape(1, N),
      batch2.astype(jnp.int32).reshape(1, N),
      Wfc, bfc.reshape(1, 1))
    return out


# speedup vs baseline: 1.1955x; 1.0351x over previous
import functools

import jax
import jax.numpy as jnp
from jax import lax
from jax.experimental import pallas as pl
from jax.experimental.pallas import tpu as pltpu
from jax.experimental.pallas import tpu_sc as plsc

N = 10000
E = 320000
D = 128
G = 64
NC = 2
NS = 16
CH = 160
K = 128
EP = NS * CH * K
SB = 32
NSB = CH // SB
NPAD = 10240
NN = 2 * N

_f32 = jnp.float32


@functools.cache
def _make_sc_degree():
    mesh = plsc.VectorSubcoreMesh(core_axis_name="c", subcore_axis_name="s")
    return functools.partial(
        pl.kernel,
        out_type=jax.ShapeDtypeStruct((NC, 1, N * 16), _f32),
        mesh=mesh,
        scratch_types=[
            pltpu.VMEM((SB, 1, K), jnp.int32),
            pltpu.VMEM((K,), _f32),
            pltpu.VMEM((2000,), _f32),
            pltpu.VMEM_SHARED((NPAD * 16,), _f32),
            pltpu.SemaphoreType.DMA,
        ],
    )(_sc_degree_body)


def _sc_degree_body(cols_hbm, deg_hbm, col_v, ones_v, zer_v, acc, semd):
    cid = lax.axis_index("c")
    sid = lax.axis_index("s")
    for i in range(K // 16):
        ones_v[pl.ds(i * 16, 16)] = jnp.ones((16,), _f32)

    def _z(i, c):
        zer_v[pl.ds(i * 16, 16)] = jnp.zeros((16,), _f32)
        return c

    lax.fori_loop(0, 2000 // 16, _z, 0)

    @pl.when(sid < 10)
    def _():
        for t in range(8):
            pltpu.sync_copy(zer_v, acc.at[pl.ds(sid * 16000 + t * 2000, 2000)])

    plsc.subcore_barrier()
    base = (cid * NS + sid) * CH

    def _body(s, c):
        pltpu.sync_copy(cols_hbm.at[pl.ds(base + s * SB, SB)], col_v)
        for m in range(SB):
            pltpu.async_copy(ones_v, acc.at[col_v.at[m, 0]], semd, add=True)
        for m in range(SB):
            pltpu.make_async_copy(ones_v, acc.at[col_v.at[m, 0]], semd).wait()
        return c

    lax.fori_loop(0, NSB, _body, 0)
    plsc.subcore_barrier()

    @pl.when(sid < 10)
    def _():
        pltpu.sync_copy(acc.at[pl.ds(sid * 16000, 16000)],
                        deg_hbm.at[cid, 0, pl.ds(sid * 16000, 16000)])


@functools.cache
def _make_sc_scatter():
    mesh = plsc.VectorSubcoreMesh(core_axis_name="c", subcore_axis_name="s")
    return functools.partial(
        pl.kernel,
        out_type=jax.ShapeDtypeStruct((NC, N, D), _f32),
        mesh=mesh,
        scratch_types=[
            pltpu.VMEM((SB, 2, K), jnp.int32),
            pltpu.VMEM((K, D), _f32),
            pltpu.VMEM((K, D), _f32),
            pltpu.VMEM_SHARED((NPAD, D), _f32),
            pltpu.SemaphoreType.DMA,
            pltpu.SemaphoreType.DMA,
        ],
    )(_sc_scatter_body)


def _sc_scatter_body(y_hbm, rc_hbm, out_hbm, rc_v, buf0, buf1, acc,
                     sem0, sem1):
    cid = lax.axis_index("c")
    sid = lax.axis_index("s")

    pltpu.sync_copy(y_hbm.at[pl.ds(cid * N + sid * 624, 624)],
                    acc.at[pl.ds(sid * 624, 624)])

    @pl.when(sid == NS - 1)
    def _():
        pltpu.sync_copy(y_hbm.at[pl.ds(cid * N + 9984, 16)],
                        acc.at[pl.ds(9984, 16)])

    plsc.subcore_barrier()
    base = (cid * NS + sid) * CH
    bufs = (buf0, buf1)
    sems = (sem0, sem1)

    def _body(s, c):
        pltpu.sync_copy(rc_hbm.at[pl.ds(base + s * SB, SB)], rc_v)
        pltpu.async_copy(y_hbm.at[rc_v.at[0, 0]], buf0, sem0)
        pltpu.async_copy(y_hbm.at[rc_v.at[1, 0]], buf1, sem1)
        for m in range(SB):
            b, sm = bufs[m % 2], sems[m % 2]
            pltpu.make_async_copy(y_hbm.at[rc_v.at[m, 0]], b, sm).wait()
            pltpu.sync_copy(b, acc.at[rc_v.at[m, 1]], add=True)
            if m + 2 < SB:
                pltpu.async_copy(y_hbm.at[rc_v.at[m + 2, 0]], b, sm)
        return c

    lax.fori_loop(0, NSB, _body, 0)
    plsc.subcore_barrier()

    pltpu.sync_copy(acc.at[pl.ds(sid * 624, 624)],
                    out_hbm.at[cid, pl.ds(sid * 624, 624)])

    @pl.when(sid == NS - 1)
    def _():
        pltpu.sync_copy(acc.at[pl.ds(9984, 16)],
                        out_hbm.at[cid, pl.ds(9984, 16)])


RB = 2000


def _tc_scale_matmul(x_ref, w_ref, deg16_ref, y_ref, dinv_ref):
    deg = jnp.sum(deg16_ref[...], axis=1, keepdims=True) + 1.0
    dinv = lax.rsqrt(deg)
    xw = jnp.dot(x_ref[...], w_ref[...], preferred_element_type=_f32)
    y_ref[...] = xw * dinv
    dinv_ref[...] = dinv


def _tc_mid(s_ref, dinv_ref, b_ref, w2_ref, y2_ref):
    dinv = dinv_ref[...]
    h = jnp.maximum(dinv * s_ref[...] + b_ref[...], 0.0)
    y2_ref[...] = jnp.dot(h, w2_ref[...], preferred_element_type=_f32) * dinv


def _tc_final(s_ref, dinv_ref, b_ref, bat1_ref, bat2_ref,
              wfc_ref, bfc_ref, o_ref):
    h = jnp.maximum(dinv_ref[...] * s_ref[...] + b_ref[...], 0.0)

    def _pool(hb, bat):
        onehot = (lax.broadcasted_iota(jnp.int32, (G, N), 0) == bat).astype(_f32)
        pooled = jnp.dot(onehot, hb, preferred_element_type=_f32)
        cnt = jnp.maximum(jnp.sum(onehot, axis=1, keepdims=True), 1.0)
        return pooled / cnt

    g1 = _pool(h[:N], bat1_ref[...])
    g2 = _pool(h[N:], bat2_ref[...])
    z = jnp.concatenate([g1, g2], axis=1)
    logit = jnp.dot(z, wfc_ref[...], preferred_element_type=_f32) + bfc_ref[0, 0]
    o_ref[...] = jax.nn.sigmoid(logit)


def kernel(x1, edge_index1, batch1, x2, edge_index2, batch2,
           W1, b1, W2, b2, Wfc, bfc):
    ei1 = edge_index1.astype(jnp.int32)
    ei2 = edge_index2.astype(jnp.int32)
    npad = EP - E
    pid = jnp.arange(npad, dtype=jnp.int32)
    pad_rows = pid % 4096
    pad_cols = N + pid % (NPAD - N)
    eid = jnp.arange(EP, dtype=jnp.int32)

    def prep(ei, base):
        rows = jnp.concatenate([ei[0] + base, pad_rows + base])
        cols = jnp.concatenate([ei[1], pad_cols])
        cols16 = cols * 16 + eid % 16
        rc = jnp.stack([rows.reshape(NS * CH, K),
                        cols.reshape(NS * CH, K)], axis=1)
        return rc, cols16.reshape(NS * CH, 1, K)

    rc1, h1 = prep(ei1, 0)
    rc2, h2 = prep(ei2, N)
    rc_sc = jnp.concatenate([rc1, rc2])
    cols16_sc = jnp.concatenate([h1, h2])

    deg16 = _make_sc_degree()(cols16_sc).reshape(NN, 16)

    x_flat = jnp.concatenate([x1, x2], axis=0)
    nb = NN // RB
    y1, dinv = pl.pallas_call(
        _tc_scale_matmul,
        grid=(nb,),
        in_specs=[pl.BlockSpec((RB, D), lambda i: (i, 0)),
                  pl.BlockSpec((D, D), lambda i: (0, 0)),
                  pl.BlockSpec((RB, 16), lambda i: (i, 0))],
        out_specs=(pl.BlockSpec((RB, D), lambda i: (i, 0)),
                   pl.BlockSpec((RB, 1), lambda i: (i, 0))),
        out_shape=(jax.ShapeDtypeStruct((NN, D), _f32),
                   jax.ShapeDtypeStruct((NN, 1), _f32)),
    )(x_flat, W1, deg16)

    s1 = _make_sc_scatter()(y1, rc_sc).reshape(NN, D)

    y2 = pl.pallas_call(
        _tc_mid,
        grid=(nb,),
        in_specs=[pl.BlockSpec((RB, D), lambda i: (i, 0)),
                  pl.BlockSpec((RB, 1), lambda i: (i, 0)),
                  pl.BlockSpec((1, D), lambda i: (0, 0)),
                  pl.BlockSpec((D, D), lambda i: (0, 0))],
        out_specs=pl.BlockSpec((RB, D), lambda i: (i, 0)),
        out_shape=jax.ShapeDtypeStruct((NN, D), _f32),
    )(s1, dinv, b1.reshape(1, D), W2)

    s2 = _make_sc_scatter()(y2, rc_sc).reshape(NN, D)

    out = pl.pallas_call(
        _tc_final,
        out_shape=jax.ShapeDtypeStruct((G, 1), _f32),
    )(s2, dinv, b2.reshape(1, D),
      batch1.astype(jnp.int32).reshape(1, N),
      batch2.astype(jnp.int32).reshape(1, N),
      Wfc, bfc.reshape(1, 1))
    return out


# speedup vs baseline: 1.2292x; 1.0282x over previous
import functools

import jax
import jax.numpy as jnp
from jax import lax
from jax.experimental import pallas as pl
from jax.experimental.pallas import tpu as pltpu
from jax.experimental.pallas import tpu_sc as plsc

N = 10000
E = 320000
D = 128
G = 64
NC = 2
NS = 16
CH = 160
K = 128
EP = NS * CH * K
SB = 32
NSB = CH // SB
SSB = 16
NSP = CH // (2 * SSB)
NPAD = 10240
NN = 2 * N

_f32 = jnp.float32


@functools.cache
def _make_sc_degree():
    mesh = plsc.VectorSubcoreMesh(core_axis_name="c", subcore_axis_name="s")
    return functools.partial(
        pl.kernel,
        out_type=jax.ShapeDtypeStruct((NC, 1, N * 16), _f32),
        mesh=mesh,
        scratch_types=[
            pltpu.VMEM((SB, 1, K), jnp.int32),
            pltpu.VMEM((K,), _f32),
            pltpu.VMEM((2000,), _f32),
            pltpu.VMEM_SHARED((NPAD * 16,), _f32),
            pltpu.SemaphoreType.DMA,
        ],
    )(_sc_degree_body)


def _sc_degree_body(cols_hbm, deg_hbm, col_v, ones_v, zer_v, acc, semd):
    cid = lax.axis_index("c")
    sid = lax.axis_index("s")
    for i in range(K // 16):
        ones_v[pl.ds(i * 16, 16)] = jnp.ones((16,), _f32)

    def _z(i, c):
        zer_v[pl.ds(i * 16, 16)] = jnp.zeros((16,), _f32)
        return c

    lax.fori_loop(0, 2000 // 16, _z, 0)

    @pl.when(sid < 10)
    def _():
        for t in range(8):
            pltpu.sync_copy(zer_v, acc.at[pl.ds(sid * 16000 + t * 2000, 2000)])

    plsc.subcore_barrier()
    base = (cid * NS + sid) * CH

    def _body(s, c):
        pltpu.sync_copy(cols_hbm.at[pl.ds(base + s * SB, SB)], col_v)
        for m in range(SB):
            pltpu.async_copy(ones_v, acc.at[col_v.at[m, 0]], semd, add=True)
        for m in range(SB):
            pltpu.make_async_copy(ones_v, acc.at[col_v.at[m, 0]], semd).wait()
        return c

    lax.fori_loop(0, NSB, _body, 0)
    plsc.subcore_barrier()

    @pl.when(sid < 10)
    def _():
        pltpu.sync_copy(acc.at[pl.ds(sid * 16000, 16000)],
                        deg_hbm.at[cid, 0, pl.ds(sid * 16000, 16000)])


@functools.cache
def _make_sc_scatter():
    mesh = plsc.VectorSubcoreMesh(core_axis_name="c", subcore_axis_name="s")
    return functools.partial(
        pl.kernel,
        out_type=jax.ShapeDtypeStruct((NC, N, D), _f32),
        mesh=mesh,
        scratch_types=[
            pltpu.VMEM((SSB, 2, K), jnp.int32),
            pltpu.VMEM((SSB, 2, K), jnp.int32),
            pltpu.VMEM((K, D), _f32),
            pltpu.VMEM((K, D), _f32),
            pltpu.VMEM_SHARED((NPAD, D), _f32),
            pltpu.SemaphoreType.DMA,
            pltpu.SemaphoreType.DMA,
            pltpu.SemaphoreType.DMA,
            pltpu.SemaphoreType.DMA,
        ],
    )(_sc_scatter_body)


def _sc_scatter_body(y_hbm, rc_hbm, out_hbm, rcA, rcB, buf0, buf1, acc,
                     sem0, sem1, semiA, semiB):
    cid = lax.axis_index("c")
    sid = lax.axis_index("s")

    pltpu.sync_copy(y_hbm.at[pl.ds(cid * N + sid * 624, 624)],
                    acc.at[pl.ds(sid * 624, 624)])

    @pl.when(sid == NS - 1)
    def _():
        pltpu.sync_copy(y_hbm.at[pl.ds(cid * N + 9984, 16)],
                        acc.at[pl.ds(9984, 16)])

    plsc.subcore_barrier()
    base = (cid * NS + sid) * CH
    bufs = (buf0, buf1)
    sems = (sem0, sem1)

    pltpu.sync_copy(rc_hbm.at[pl.ds(base, SSB)], rcA)
    pltpu.async_copy(y_hbm.at[rcA.at[0, 0]], buf0, sem0)
    pltpu.async_copy(y_hbm.at[rcA.at[1, 0]], buf1, sem1)

    def _phase(rc_cur, rc_nxt, semi_nxt, sb_nxt, last):
        @pl.when(jnp.logical_not(last))
        def _():
            pltpu.async_copy(rc_hbm.at[pl.ds(base + sb_nxt * SSB, SSB)],
                             rc_nxt, semi_nxt)
        for m in range(SSB):
            b, sm = bufs[m % 2], sems[m % 2]
            pltpu.make_async_copy(y_hbm.at[rc_cur.at[m, 0]], b, sm).wait()
            pltpu.sync_copy(b, acc.at[rc_cur.at[m, 1]], add=True)
            if m + 2 < SSB:
                pltpu.async_copy(y_hbm.at[rc_cur.at[m + 2, 0]], b, sm)
            elif m == SSB - 2:
                @pl.when(jnp.logical_not(last))
                def _():
                    pltpu.make_async_copy(rc_hbm.at[pl.ds(base, SSB)],
                                          rc_nxt, semi_nxt).wait()
                    pltpu.async_copy(y_hbm.at[rc_nxt.at[0, 0]], b, sm)
            else:
                @pl.when(jnp.logical_not(last))
                def _():
                    pltpu.async_copy(y_hbm.at[rc_nxt.at[1, 0]], b, sm)

    def _pair(t, c):
        _phase(rcA, rcB, semiB, 2 * t + 1, jnp.bool_(False))
        _phase(rcB, rcA, semiA, 2 * t + 2, t >= NSP - 1)
        return c

    lax.fori_loop(0, NSP, _pair, 0)
    plsc.subcore_barrier()

    pltpu.sync_copy(acc.at[pl.ds(sid * 624, 624)],
                    out_hbm.at[cid, pl.ds(sid * 624, 624)])

    @pl.when(sid == NS - 1)
    def _():
        pltpu.sync_copy(acc.at[pl.ds(9984, 16)],
                        out_hbm.at[cid, pl.ds(9984, 16)])


RB = 2000


def _tc_scale_matmul(x_ref, w_ref, deg16_ref, y_ref, dinv_ref):
    deg = jnp.sum(deg16_ref[...], axis=1, keepdims=True) + 1.0
    dinv = lax.rsqrt(deg)
    xw = jnp.dot(x_ref[...], w_ref[...], preferred_element_type=_f32)
    y_ref[...] = xw * dinv
    dinv_ref[...] = dinv


def _tc_mid(s_ref, dinv_ref, b_ref, w2_ref, y2_ref):
    dinv = dinv_ref[...]
    h = jnp.maximum(dinv * s_ref[...] + b_ref[...], 0.0)
    y2_ref[...] = jnp.dot(h, w2_ref[...], preferred_element_type=_f32) * dinv


def _tc_final(s_ref, dinv_ref, b_ref, bat1_ref, bat2_ref,
              wfc_ref, bfc_ref, o_ref):
    h = jnp.maximum(dinv_ref[...] * s_ref[...] + b_ref[...], 0.0)

    def _pool(hb, bat):
        onehot = (lax.broadcasted_iota(jnp.int32, (G, N), 0) == bat).astype(_f32)
        pooled = jnp.dot(onehot, hb, preferred_element_type=_f32)
        cnt = jnp.maximum(jnp.sum(onehot, axis=1, keepdims=True), 1.0)
        return pooled / cnt

    g1 = _pool(h[:N], bat1_ref[...])
    g2 = _pool(h[N:], bat2_ref[...])
    z = jnp.concatenate([g1, g2], axis=1)
    logit = jnp.dot(z, wfc_ref[...], preferred_element_type=_f32) + bfc_ref[0, 0]
    o_ref[...] = jax.nn.sigmoid(logit)


def kernel(x1, edge_index1, batch1, x2, edge_index2, batch2,
           W1, b1, W2, b2, Wfc, bfc):
    ei1 = edge_index1.astype(jnp.int32)
    ei2 = edge_index2.astype(jnp.int32)
    npad = EP - E
    pid = jnp.arange(npad, dtype=jnp.int32)
    pad_rows = pid % 4096
    pad_cols = N + pid % (NPAD - N)
    eid = jnp.arange(EP, dtype=jnp.int32)

    def prep(ei, base):
        rows = jnp.concatenate([ei[0] + base, pad_rows + base])
        cols = jnp.concatenate([ei[1], pad_cols])
        cols16 = cols * 16 + eid % 16
        rc = jnp.stack([rows.reshape(NS * CH, K),
                        cols.reshape(NS * CH, K)], axis=1)
        return rc, cols16.reshape(NS * CH, 1, K)

    rc1, h1 = prep(ei1, 0)
    rc2, h2 = prep(ei2, N)
    rc_sc = jnp.concatenate([rc1, rc2])
    cols16_sc = jnp.concatenate([h1, h2])

    deg16 = _make_sc_degree()(cols16_sc).reshape(NN, 16)

    x_flat = jnp.concatenate([x1, x2], axis=0)
    nb = NN // RB
    y1, dinv = pl.pallas_call(
        _tc_scale_matmul,
        grid=(nb,),
        in_specs=[pl.BlockSpec((RB, D), lambda i: (i, 0)),
                  pl.BlockSpec((D, D), lambda i: (0, 0)),
                  pl.BlockSpec((RB, 16), lambda i: (i, 0))],
        out_specs=(pl.BlockSpec((RB, D), lambda i: (i, 0)),
                   pl.BlockSpec((RB, 1), lambda i: (i, 0))),
        out_shape=(jax.ShapeDtypeStruct((NN, D), _f32),
                   jax.ShapeDtypeStruct((NN, 1), _f32)),
    )(x_flat, W1, deg16)

    s1 = _make_sc_scatter()(y1, rc_sc).reshape(NN, D)

    y2 = pl.pallas_call(
        _tc_mid,
        grid=(nb,),
        in_specs=[pl.BlockSpec((RB, D), lambda i: (i, 0)),
                  pl.BlockSpec((RB, 1), lambda i: (i, 0)),
                  pl.BlockSpec((1, D), lambda i: (0, 0)),
                  pl.BlockSpec((D, D), lambda i: (0, 0))],
        out_specs=pl.BlockSpec((RB, D), lambda i: (i, 0)),
        out_shape=jax.ShapeDtypeStruct((NN, D), _f32),
    )(s1, dinv, b1.reshape(1, D), W2)

    s2 = _make_sc_scatter()(y2, rc_sc).reshape(NN, D)

    out = pl.pallas_call(
        _tc_final,
        out_shape=jax.ShapeDtypeStruct((G, 1), _f32),
    )(s2, dinv, b2.reshape(1, D),
      batch1.astype(jnp.int32).reshape(1, N),
      batch2.astype(jnp.int32).reshape(1, N),
      Wfc, bfc.reshape(1, 1))
    return out
